# Initial kernel scaffold; baseline (speedup 1.0000x reference)
import functools

import jax
import jax.numpy as jnp
from jax import lax
from jax.experimental import pallas as pl
from jax.experimental.pallas import tpu as pltpu
from jax.experimental.pallas import tpu_sc as plsc

N_NODES = 10000
N_EDGES = 320000
D_IN = 128
D_HID = 16
D_OUT = 40
D_OUT_PAD = 48

NC = 2
NS = 16
NW = NC * NS

CHUNK = 128
EPW = 10112
NCHUNK = EPW // CHUNK
E_PAD = EPW * NW
ACC_ROWS = 10112
ROWS_PER_TILE = ACC_ROWS // NS
PAD_DST = N_NODES

_sc_mesh = plsc.VectorSubcoreMesh(core_axis_name="c", subcore_axis_name="s")


def _worker_id():
    return lax.axis_index("s") * NC + lax.axis_index("c")


@functools.partial(
    pl.kernel,
    out_type=jax.ShapeDtypeStruct((NC, ACC_ROWS, 16), jnp.float32),
    mesh=_sc_mesh,
    scratch_types=[
        pltpu.VMEM((CHUNK,), jnp.int32),
        pltpu.VMEM((CHUNK, 16), jnp.float32),
    ],
)
def _sc_degree(dst_hbm, ones_hbm, zeros_hbm, out_hbm, idx_v, ones_v):
    cid = lax.axis_index("c")
    sid = lax.axis_index("s")
    wid = _worker_id()

    def scoped(acc):
        row0 = sid * ROWS_PER_TILE
        pltpu.sync_copy(
            zeros_hbm.at[pl.ds(row0, ROWS_PER_TILE)],
            acc.at[pl.ds(row0, ROWS_PER_TILE)],
        )
        pltpu.sync_copy(ones_hbm, ones_v)
        plsc.subcore_barrier()

        base = wid * EPW

        def body(i, carry):
            pltpu.sync_copy(dst_hbm.at[pl.ds(base + i * CHUNK, CHUNK)], idx_v)
            pltpu.sync_copy(ones_v, acc.at[idx_v], add=True)
            return carry

        lax.fori_loop(0, NCHUNK, body, 0)
        plsc.subcore_barrier()
        pltpu.sync_copy(
            acc.at[pl.ds(row0, ROWS_PER_TILE)],
            out_hbm.at[cid, pl.ds(row0, ROWS_PER_TILE)],
        )

    pl.run_scoped(
        scoped, pltpu.MemorySpace.VMEM_SHARED((ACC_ROWS, 16), jnp.float32)
    )


def _make_sc_aggregate(feat):
    @functools.partial(
        pl.kernel,
        out_type=jax.ShapeDtypeStruct((NC, ACC_ROWS, feat), jnp.float32),
        mesh=_sc_mesh,
        scratch_types=[
            pltpu.VMEM((CHUNK,), jnp.int32),
            pltpu.VMEM((CHUNK,), jnp.int32),
            pltpu.VMEM((CHUNK, feat), jnp.float32),
            pltpu.SemaphoreType.DMA,
        ],
    )
    def agg(g_hbm, src_hbm, dst_hbm, zeros_hbm, out_hbm, sidx_v, didx_v,
            msg_v, sem):
        cid = lax.axis_index("c")
        sid = lax.axis_index("s")
        wid = _worker_id()

        def scoped(acc):
            row0 = sid * ROWS_PER_TILE
            pltpu.sync_copy(
                zeros_hbm.at[pl.ds(row0, ROWS_PER_TILE)],
                acc.at[pl.ds(row0, ROWS_PER_TILE)],
            )
            plsc.subcore_barrier()

            base = wid * EPW

            def body(i, carry):
                off = base + i * CHUNK
                pltpu.sync_copy(src_hbm.at[pl.ds(off, CHUNK)], sidx_v)
                pltpu.sync_copy(dst_hbm.at[pl.ds(off, CHUNK)], didx_v)
                pltpu.async_copy(g_hbm.at[sidx_v], msg_v, sem).wait()
                pltpu.sync_copy(msg_v, acc.at[didx_v], add=True)
                return carry

            lax.fori_loop(0, NCHUNK, body, 0)
            plsc.subcore_barrier()
            pltpu.sync_copy(
                acc.at[pl.ds(row0, ROWS_PER_TILE)],
                out_hbm.at[cid, pl.ds(row0, ROWS_PER_TILE)],
            )

        pl.run_scoped(
            scoped, pltpu.MemorySpace.VMEM_SHARED((ACC_ROWS, feat), jnp.float32)
        )

    return agg


_sc_agg16 = _make_sc_aggregate(D_HID)
_sc_agg48 = _make_sc_aggregate(D_OUT_PAD)


ROW_BLK = 1000
GRID = N_NODES // ROW_BLK


def _tc1_body(x_ref, w1_ref, degp_ref, g1_ref, dinv_ref):
    deg = degp_ref[0, :, 0:1] + degp_ref[1, :, 0:1] + 1.0
    dinv = lax.rsqrt(deg)
    h = jnp.dot(x_ref[...], w1_ref[...], preferred_element_type=jnp.float32)
    g1_ref[...] = h * dinv
    dinv_ref[...] = dinv


def _tc1(x, w1, degp):
    return pl.pallas_call(
        _tc1_body,
        grid=(GRID,),
        in_specs=[
            pl.BlockSpec((ROW_BLK, D_IN), lambda i: (i, 0)),
            pl.BlockSpec((D_IN, D_HID), lambda i: (0, 0)),
            pl.BlockSpec((NC, ROW_BLK, 16), lambda i: (0, i, 0)),
        ],
        out_specs=[
            pl.BlockSpec((ROW_BLK, D_HID), lambda i: (i, 0)),
            pl.BlockSpec((ROW_BLK, 1), lambda i: (i, 0)),
        ],
        out_shape=[
            jax.ShapeDtypeStruct((N_NODES, D_HID), jnp.float32),
            jax.ShapeDtypeStruct((N_NODES, 1), jnp.float32),
        ],
    )(x, w1, degp)


def _tc2_body(aggp_ref, g1_ref, dinv_ref, b1_ref, w2_ref, g2_ref):
    s = aggp_ref[0] + aggp_ref[1] + g1_ref[...]
    out1 = jnp.maximum(dinv_ref[...] * s + b1_ref[...], 0.0)
    h2 = jnp.dot(out1, w2_ref[...], preferred_element_type=jnp.float32)
    g2_ref[...] = h2 * dinv_ref[...]


def _tc2(aggp, g1, dinv, b1, w2p):
    return pl.pallas_call(
        _tc2_body,
        grid=(GRID,),
        in_specs=[
            pl.BlockSpec((NC, ROW_BLK, D_HID), lambda i: (0, i, 0)),
            pl.BlockSpec((ROW_BLK, D_HID), lambda i: (i, 0)),
            pl.BlockSpec((ROW_BLK, 1), lambda i: (i, 0)),
            pl.BlockSpec((1, D_HID), lambda i: (0, 0)),
            pl.BlockSpec((D_HID, D_OUT_PAD), lambda i: (0, 0)),
        ],
        out_specs=pl.BlockSpec((ROW_BLK, D_OUT_PAD), lambda i: (i, 0)),
        out_shape=jax.ShapeDtypeStruct((N_NODES, D_OUT_PAD), jnp.float32),
    )(aggp, g1, dinv, b1, w2p)


def _tc3_body(aggp_ref, g2_ref, dinv_ref, b2_ref, out_ref):
    s = aggp_ref[0] + aggp_ref[1] + g2_ref[...]
    res = dinv_ref[...] * s + b2_ref[...]
    out_ref[...] = res[:, :D_OUT]


def _tc3(aggp, g2, dinv, b2p):
    return pl.pallas_call(
        _tc3_body,
        grid=(GRID,),
        in_specs=[
            pl.BlockSpec((NC, ROW_BLK, D_OUT_PAD), lambda i: (0, i, 0)),
            pl.BlockSpec((ROW_BLK, D_OUT_PAD), lambda i: (i, 0)),
            pl.BlockSpec((ROW_BLK, 1), lambda i: (i, 0)),
            pl.BlockSpec((1, D_OUT_PAD), lambda i: (0, 0)),
        ],
        out_specs=pl.BlockSpec((ROW_BLK, D_OUT), lambda i: (i, 0)),
        out_shape=jax.ShapeDtypeStruct((N_NODES, D_OUT), jnp.float32),
    )(aggp, g2, dinv, b2p)


def kernel(x, edge_index, W1, b1, W2, b2):
    src = edge_index[0].astype(jnp.int32)
    dst = edge_index[1].astype(jnp.int32)
    npad = E_PAD - N_EDGES
    srcp = jnp.concatenate([src, jnp.zeros((npad,), jnp.int32)])
    dstp = jnp.concatenate([dst, jnp.full((npad,), PAD_DST, jnp.int32)])

    ones16 = jnp.ones((CHUNK, 16), jnp.float32)
    zeros16 = jnp.zeros((ACC_ROWS, 16), jnp.float32)
    zeros48 = jnp.zeros((ACC_ROWS, D_OUT_PAD), jnp.float32)

    degp = _sc_degree(dstp, ones16, zeros16)
    degp = degp[:, :N_NODES, :]

    g1, dinv = _tc1(x, W1, degp)

    agg1 = _sc_agg16(g1, srcp, dstp, zeros16)
    agg1 = agg1[:, :N_NODES, :]

    w2p = jnp.pad(W2, ((0, 0), (0, D_OUT_PAD - D_OUT)))
    b1r = b1.reshape(1, D_HID)
    b2p = jnp.pad(b2, (0, D_OUT_PAD - D_OUT)).reshape(1, D_OUT_PAD)

    g2 = _tc2(agg1, g1, dinv, b1r, w2p)

    agg2 = _sc_agg48(g2, srcp, dstp, zeros48)
    agg2 = agg2[:, :N_NODES, :]

    return _tc3(agg2, g2, dinv, b2p)


# speedup vs baseline: 17.4910x; 17.4910x over previous
import functools

import jax
import jax.numpy as jnp
from jax import lax
from jax.experimental import pallas as pl
from jax.experimental.pallas import tpu as pltpu
from jax.experimental.pallas import tpu_sc as plsc

N_NODES = 10000
N_EDGES = 320000
D_IN = 128
D_HID = 16
D_OUT = 40
D_OUT_PAD = 48

NC = 2
NS = 16
NW = NC * NS

CHUNK = 128
EPW = 10112
NCHUNK = EPW // CHUNK
E_PAD = EPW * NW
ACC_ROWS = 10112
ROWS_PER_TILE = ACC_ROWS // NS
PAD_DST = N_NODES

_sc_mesh = plsc.VectorSubcoreMesh(core_axis_name="c", subcore_axis_name="s")
_sc_params = pltpu.CompilerParams(use_tc_tiling_on_sc=False)


def _worker_id():
    return lax.axis_index("s") * NC + lax.axis_index("c")


@functools.partial(
    pl.kernel,
    out_type=jax.ShapeDtypeStruct((NC, ACC_ROWS, 16), jnp.float32),
    mesh=_sc_mesh,
    compiler_params=_sc_params,
    scratch_types=[
        pltpu.VMEM((CHUNK,), jnp.int32),
        pltpu.VMEM((CHUNK, 16), jnp.float32),
        pltpu.VMEM_SHARED((ACC_ROWS, 16), jnp.float32),
    ],
)
def _sc_degree(dst_hbm, ones_hbm, zeros_hbm, out_hbm, idx_v, ones_v, acc):
    cid = lax.axis_index("c")
    sid = lax.axis_index("s")
    wid = _worker_id()

    row0 = sid * ROWS_PER_TILE
    pltpu.sync_copy(
        zeros_hbm.at[pl.ds(row0, ROWS_PER_TILE)],
        acc.at[pl.ds(row0, ROWS_PER_TILE)],
    )
    pltpu.sync_copy(ones_hbm, ones_v)
    plsc.subcore_barrier()

    base = wid * EPW

    def body(i, carry):
        pltpu.sync_copy(dst_hbm.at[pl.ds(base + i * CHUNK, CHUNK)], idx_v)
        pltpu.sync_copy(ones_v, acc.at[idx_v], add=True)
        return carry

    lax.fori_loop(0, NCHUNK, body, 0)
    plsc.subcore_barrier()
    pltpu.sync_copy(
        acc.at[pl.ds(row0, ROWS_PER_TILE)],
        out_hbm.at[cid, pl.ds(row0, ROWS_PER_TILE)],
    )


def _make_sc_aggregate(feat):
    @functools.partial(
        pl.kernel,
        out_type=jax.ShapeDtypeStruct((NC, ACC_ROWS, feat), jnp.float32),
        mesh=_sc_mesh,
        compiler_params=_sc_params,
        scratch_types=[
            pltpu.VMEM((CHUNK,), jnp.int32),
            pltpu.VMEM((CHUNK,), jnp.int32),
            pltpu.VMEM((CHUNK, feat), jnp.float32),
            pltpu.SemaphoreType.DMA,
            pltpu.VMEM_SHARED((ACC_ROWS, feat), jnp.float32),
        ],
    )
    def agg(g_hbm, src_hbm, dst_hbm, zeros_hbm, out_hbm, sidx_v, didx_v,
            msg_v, sem, acc):
        cid = lax.axis_index("c")
        sid = lax.axis_index("s")
        wid = _worker_id()

        row0 = sid * ROWS_PER_TILE
        pltpu.sync_copy(
            zeros_hbm.at[pl.ds(row0, ROWS_PER_TILE)],
            acc.at[pl.ds(row0, ROWS_PER_TILE)],
        )
        plsc.subcore_barrier()

        base = wid * EPW

        def body(i, carry):
            off = base + i * CHUNK
            pltpu.sync_copy(src_hbm.at[pl.ds(off, CHUNK)], sidx_v)
            pltpu.sync_copy(dst_hbm.at[pl.ds(off, CHUNK)], didx_v)
            pltpu.async_copy(g_hbm.at[sidx_v], msg_v, sem).wait()
            pltpu.sync_copy(msg_v, acc.at[didx_v], add=True)
            return carry

        lax.fori_loop(0, NCHUNK, body, 0)
        plsc.subcore_barrier()
        pltpu.sync_copy(
            acc.at[pl.ds(row0, ROWS_PER_TILE)],
            out_hbm.at[cid, pl.ds(row0, ROWS_PER_TILE)],
        )

    return agg


_sc_agg16 = _make_sc_aggregate(D_HID)
_sc_agg48 = _make_sc_aggregate(D_OUT_PAD)


ROW_BLK = 1000
GRID = N_NODES // ROW_BLK


def _tc1_body(x_ref, w1_ref, degp_ref, g1_ref, dinv_ref):
    deg = degp_ref[0, :, 0:1] + degp_ref[1, :, 0:1] + 1.0
    dinv = lax.rsqrt(deg)
    h = jnp.dot(x_ref[...], w1_ref[...], preferred_element_type=jnp.float32)
    g1_ref[...] = h * dinv
    dinv_ref[...] = dinv


def _tc1(x, w1, degp):
    return pl.pallas_call(
        _tc1_body,
        grid=(GRID,),
        in_specs=[
            pl.BlockSpec((ROW_BLK, D_IN), lambda i: (i, 0)),
            pl.BlockSpec((D_IN, D_HID), lambda i: (0, 0)),
            pl.BlockSpec((NC, ROW_BLK, 16), lambda i: (0, i, 0)),
        ],
        out_specs=[
            pl.BlockSpec((ROW_BLK, D_HID), lambda i: (i, 0)),
            pl.BlockSpec((ROW_BLK, 1), lambda i: (i, 0)),
        ],
        out_shape=[
            jax.ShapeDtypeStruct((N_NODES, D_HID), jnp.float32),
            jax.ShapeDtypeStruct((N_NODES, 1), jnp.float32),
        ],
    )(x, w1, degp)


def _tc2_body(aggp_ref, g1_ref, dinv_ref, b1_ref, w2_ref, g2_ref):
    s = aggp_ref[0] + aggp_ref[1] + g1_ref[...]
    out1 = jnp.maximum(dinv_ref[...] * s + b1_ref[...], 0.0)
    h2 = jnp.dot(out1, w2_ref[...], preferred_element_type=jnp.float32)
    g2_ref[...] = h2 * dinv_ref[...]


def _tc2(aggp, g1, dinv, b1, w2p):
    return pl.pallas_call(
        _tc2_body,
        grid=(GRID,),
        in_specs=[
            pl.BlockSpec((NC, ROW_BLK, D_HID), lambda i: (0, i, 0)),
            pl.BlockSpec((ROW_BLK, D_HID), lambda i: (i, 0)),
            pl.BlockSpec((ROW_BLK, 1), lambda i: (i, 0)),
            pl.BlockSpec((1, D_HID), lambda i: (0, 0)),
            pl.BlockSpec((D_HID, D_OUT_PAD), lambda i: (0, 0)),
        ],
        out_specs=pl.BlockSpec((ROW_BLK, D_OUT_PAD), lambda i: (i, 0)),
        out_shape=jax.ShapeDtypeStruct((N_NODES, D_OUT_PAD), jnp.float32),
    )(aggp, g1, dinv, b1, w2p)


def _tc3_body(aggp_ref, g2_ref, dinv_ref, b2_ref, out_ref):
    s = aggp_ref[0] + aggp_ref[1] + g2_ref[...]
    res = dinv_ref[...] * s + b2_ref[...]
    out_ref[...] = res[:, :D_OUT]


def _tc3(aggp, g2, dinv, b2p):
    return pl.pallas_call(
        _tc3_body,
        grid=(GRID,),
        in_specs=[
            pl.BlockSpec((NC, ROW_BLK, D_OUT_PAD), lambda i: (0, i, 0)),
            pl.BlockSpec((ROW_BLK, D_OUT_PAD), lambda i: (i, 0)),
            pl.BlockSpec((ROW_BLK, 1), lambda i: (i, 0)),
            pl.BlockSpec((1, D_OUT_PAD), lambda i: (0, 0)),
        ],
        out_specs=pl.BlockSpec((ROW_BLK, D_OUT), lambda i: (i, 0)),
        out_shape=jax.ShapeDtypeStruct((N_NODES, D_OUT), jnp.float32),
    )(aggp, g2, dinv, b2p)


def kernel(x, edge_index, W1, b1, W2, b2):
    src = edge_index[0].astype(jnp.int32)
    dst = edge_index[1].astype(jnp.int32)
    npad = E_PAD - N_EDGES
    srcp = jnp.concatenate([src, jnp.zeros((npad,), jnp.int32)])
    dstp = jnp.concatenate([dst, jnp.full((npad,), PAD_DST, jnp.int32)])

    ones16 = jnp.ones((CHUNK, 16), jnp.float32)
    zeros16 = jnp.zeros((ACC_ROWS, 16), jnp.float32)
    zeros48 = jnp.zeros((ACC_ROWS, D_OUT_PAD), jnp.float32)

    degp = _sc_degree(dstp, ones16, zeros16)
    degp = degp[:, :N_NODES, :]

    g1, dinv = _tc1(x, W1, degp)

    agg1 = _sc_agg16(g1, srcp, dstp, zeros16)
    agg1 = agg1[:, :N_NODES, :]

    w2p = jnp.pad(W2, ((0, 0), (0, D_OUT_PAD - D_OUT)))
    b1r = b1.reshape(1, D_HID)
    b2p = jnp.pad(b2, (0, D_OUT_PAD - D_OUT)).reshape(1, D_OUT_PAD)

    g2 = _tc2(agg1, g1, dinv, b1r, w2p)

    agg2 = _sc_agg48(g2, srcp, dstp, zeros48)
    agg2 = agg2[:, :N_NODES, :]

    return _tc3(agg2, g2, dinv, b2p)


# speedup vs baseline: 23.2883x; 1.3314x over previous
import functools

import jax
import jax.numpy as jnp
from jax import lax
from jax.experimental import pallas as pl
from jax.experimental.pallas import tpu as pltpu
from jax.experimental.pallas import tpu_sc as plsc

N_NODES = 10000
N_EDGES = 320000
D_IN = 128
D_HID = 16
D_OUT = 40
D_OUT_PAD = 48

NC = 2
NS = 16
NW = NC * NS

CHUNK = 128
NCHUNK = 80
NBUF = 8
EPW = NCHUNK * CHUNK
E_PAD = EPW * NW
ACC_ROWS = 10112
ROWS_PER_TILE = ACC_ROWS // NS
PAD_DST = N_NODES

_sc_mesh = plsc.VectorSubcoreMesh(core_axis_name="c", subcore_axis_name="s")
_sc_params = pltpu.CompilerParams(use_tc_tiling_on_sc=False)


def _worker_id():
    return lax.axis_index("s") * NC + lax.axis_index("c")


@functools.partial(
    pl.kernel,
    out_type=jax.ShapeDtypeStruct((NC, ACC_ROWS, 16), jnp.float32),
    mesh=_sc_mesh,
    compiler_params=_sc_params,
    scratch_types=[
        pltpu.VMEM((NCHUNK, CHUNK), jnp.int32),
        pltpu.VMEM((CHUNK, 16), jnp.float32),
        pltpu.VMEM_SHARED((ACC_ROWS, 16), jnp.float32),
        pltpu.SemaphoreType.DMA,
    ],
)
def _sc_degree(dst_hbm, ones_hbm, zeros_hbm, out_hbm, didx_v, ones_v, acc,
               ssem):
    cid = lax.axis_index("c")
    sid = lax.axis_index("s")
    wid = _worker_id()

    row0 = sid * ROWS_PER_TILE
    pltpu.sync_copy(
        zeros_hbm.at[pl.ds(row0, ROWS_PER_TILE)],
        acc.at[pl.ds(row0, ROWS_PER_TILE)],
    )
    pltpu.sync_copy(dst_hbm.at[pl.ds(wid * NCHUNK, NCHUNK)], didx_v)
    pltpu.sync_copy(ones_hbm, ones_v)
    plsc.subcore_barrier()

    def body(t, carry):
        base = t * NBUF
        descs = [
            pltpu.async_copy(ones_v, acc.at[didx_v.at[base + j]], ssem,
                             add=True)
            for j in range(NBUF)
        ]
        for d in descs:
            d.wait()
        return carry

    lax.fori_loop(0, NCHUNK // NBUF, body, 0)
    plsc.subcore_barrier()
    pltpu.sync_copy(
        acc.at[pl.ds(row0, ROWS_PER_TILE)],
        out_hbm.at[cid, pl.ds(row0, ROWS_PER_TILE)],
    )


def _make_sc_aggregate(feat):
    @functools.partial(
        pl.kernel,
        out_type=jax.ShapeDtypeStruct((NC, ACC_ROWS, feat), jnp.float32),
        mesh=_sc_mesh,
        compiler_params=_sc_params,
        scratch_types=[
            pltpu.VMEM((NCHUNK, CHUNK), jnp.int32),
            pltpu.VMEM((NCHUNK, CHUNK), jnp.int32),
            pltpu.VMEM((NBUF, CHUNK, feat), jnp.float32),
            pltpu.SemaphoreType.DMA,
            pltpu.SemaphoreType.DMA,
            pltpu.VMEM_SHARED((ACC_ROWS, feat), jnp.float32),
        ],
    )
    def agg(g_hbm, src_hbm, dst_hbm, zeros_hbm, out_hbm, sidx_v, didx_v,
            msg_v, gsem, ssem, acc):
        cid = lax.axis_index("c")
        sid = lax.axis_index("s")
        wid = _worker_id()

        row0 = sid * ROWS_PER_TILE
        pltpu.sync_copy(
            zeros_hbm.at[pl.ds(row0, ROWS_PER_TILE)],
            acc.at[pl.ds(row0, ROWS_PER_TILE)],
        )
        pltpu.sync_copy(src_hbm.at[pl.ds(wid * NCHUNK, NCHUNK)], sidx_v)
        pltpu.sync_copy(dst_hbm.at[pl.ds(wid * NCHUNK, NCHUNK)], didx_v)
        plsc.subcore_barrier()

        def body(t, carry):
            base = t * NBUF
            gd = [
                pltpu.async_copy(g_hbm.at[sidx_v.at[base + j]], msg_v.at[j],
                                 gsem)
                for j in range(NBUF)
            ]
            for d in gd:
                d.wait()
            sd = [
                pltpu.async_copy(msg_v.at[j], acc.at[didx_v.at[base + j]],
                                 ssem, add=True)
                for j in range(NBUF)
            ]
            for d in sd:
                d.wait()
            return carry

        lax.fori_loop(0, NCHUNK // NBUF, body, 0)
        plsc.subcore_barrier()
        pltpu.sync_copy(
            acc.at[pl.ds(row0, ROWS_PER_TILE)],
            out_hbm.at[cid, pl.ds(row0, ROWS_PER_TILE)],
        )

    return agg


_sc_agg16 = _make_sc_aggregate(D_HID)
_sc_agg48 = _make_sc_aggregate(D_OUT_PAD)


ROW_BLK = 1000
GRID = N_NODES // ROW_BLK


def _tc1_body(x_ref, w1_ref, degp_ref, g1_ref, dinv_ref):
    deg = degp_ref[0, :, 0:1] + degp_ref[1, :, 0:1] + 1.0
    dinv = lax.rsqrt(deg)
    h = jnp.dot(x_ref[...], w1_ref[...], preferred_element_type=jnp.float32)
    g1_ref[...] = h * dinv
    dinv_ref[...] = dinv


def _tc1(x, w1, degp):
    return pl.pallas_call(
        _tc1_body,
        grid=(GRID,),
        in_specs=[
            pl.BlockSpec((ROW_BLK, D_IN), lambda i: (i, 0)),
            pl.BlockSpec((D_IN, D_HID), lambda i: (0, 0)),
            pl.BlockSpec((NC, ROW_BLK, 16), lambda i: (0, i, 0)),
        ],
        out_specs=[
            pl.BlockSpec((ROW_BLK, D_HID), lambda i: (i, 0)),
            pl.BlockSpec((ROW_BLK, 1), lambda i: (i, 0)),
        ],
        out_shape=[
            jax.ShapeDtypeStruct((N_NODES, D_HID), jnp.float32),
            jax.ShapeDtypeStruct((N_NODES, 1), jnp.float32),
        ],
    )(x, w1, degp)


def _tc2_body(aggp_ref, g1_ref, dinv_ref, b1_ref, w2_ref, g2_ref):
    s = aggp_ref[0] + aggp_ref[1] + g1_ref[...]
    out1 = jnp.maximum(dinv_ref[...] * s + b1_ref[...], 0.0)
    h2 = jnp.dot(out1, w2_ref[...], preferred_element_type=jnp.float32)
    g2_ref[...] = h2 * dinv_ref[...]


def _tc2(aggp, g1, dinv, b1, w2p):
    return pl.pallas_call(
        _tc2_body,
        grid=(GRID,),
        in_specs=[
            pl.BlockSpec((NC, ROW_BLK, D_HID), lambda i: (0, i, 0)),
            pl.BlockSpec((ROW_BLK, D_HID), lambda i: (i, 0)),
            pl.BlockSpec((ROW_BLK, 1), lambda i: (i, 0)),
            pl.BlockSpec((1, D_HID), lambda i: (0, 0)),
            pl.BlockSpec((D_HID, D_OUT_PAD), lambda i: (0, 0)),
        ],
        out_specs=pl.BlockSpec((ROW_BLK, D_OUT_PAD), lambda i: (i, 0)),
        out_shape=jax.ShapeDtypeStruct((N_NODES, D_OUT_PAD), jnp.float32),
    )(aggp, g1, dinv, b1, w2p)


def _tc3_body(aggp_ref, g2_ref, dinv_ref, b2_ref, out_ref):
    s = aggp_ref[0] + aggp_ref[1] + g2_ref[...]
    res = dinv_ref[...] * s + b2_ref[...]
    out_ref[...] = res[:, :D_OUT]


def _tc3(aggp, g2, dinv, b2p):
    return pl.pallas_call(
        _tc3_body,
        grid=(GRID,),
        in_specs=[
            pl.BlockSpec((NC, ROW_BLK, D_OUT_PAD), lambda i: (0, i, 0)),
            pl.BlockSpec((ROW_BLK, D_OUT_PAD), lambda i: (i, 0)),
            pl.BlockSpec((ROW_BLK, 1), lambda i: (i, 0)),
            pl.BlockSpec((1, D_OUT_PAD), lambda i: (0, 0)),
        ],
        out_specs=pl.BlockSpec((ROW_BLK, D_OUT), lambda i: (i, 0)),
        out_shape=jax.ShapeDtypeStruct((N_NODES, D_OUT), jnp.float32),
    )(aggp, g2, dinv, b2p)


def kernel(x, edge_index, W1, b1, W2, b2):
    src = edge_index[0].astype(jnp.int32)
    dst = edge_index[1].astype(jnp.int32)
    npad = E_PAD - N_EDGES
    srcp = jnp.concatenate([src, jnp.zeros((npad,), jnp.int32)])
    srcp = srcp.reshape(NW * NCHUNK, CHUNK)
    dstp = jnp.concatenate([dst, jnp.full((npad,), PAD_DST, jnp.int32)])
    dstp = dstp.reshape(NW * NCHUNK, CHUNK)

    ones16 = jnp.ones((CHUNK, 16), jnp.float32)
    zeros16 = jnp.zeros((ACC_ROWS, 16), jnp.float32)
    zeros48 = jnp.zeros((ACC_ROWS, D_OUT_PAD), jnp.float32)

    degp = _sc_degree(dstp, ones16, zeros16)
    degp = degp[:, :N_NODES, :]

    g1, dinv = _tc1(x, W1, degp)

    agg1 = _sc_agg16(g1, srcp, dstp, zeros16)
    agg1 = agg1[:, :N_NODES, :]

    w2p = jnp.pad(W2, ((0, 0), (0, D_OUT_PAD - D_OUT)))
    b1r = b1.reshape(1, D_HID)
    b2p = jnp.pad(b2, (0, D_OUT_PAD - D_OUT)).reshape(1, D_OUT_PAD)

    g2 = _tc2(agg1, g1, dinv, b1r, w2p)

    agg2 = _sc_agg48(g2, srcp, dstp, zeros48)
    agg2 = agg2[:, :N_NODES, :]

    return _tc3(agg2, g2, dinv, b2p)


# speedup vs baseline: 53.6252x; 2.3027x over previous
import functools

import jax
import jax.numpy as jnp
from jax import lax
from jax.experimental import pallas as pl
from jax.experimental.pallas import tpu as pltpu
from jax.experimental.pallas import tpu_sc as plsc

N_NODES = 10000
N_EDGES = 320000
D_IN = 128
D_HID = 16
D_OUT = 40

NC = 2
NS = 16

CHUNK = 128
STRIPE = 160
NCH0 = 104
NCH1 = STRIPE - NCH0
K = 4
NBUF = 2 * K
N_CHUNKS = NS * STRIPE
E_PAD = N_CHUNKS * CHUNK
ACC_ROWS = 10112
ROWS_PER_TILE = ACC_ROWS // NS
PACK_PER_TILE = ROWS_PER_TILE * D_HID // 128
PAD_DST = N_NODES

_sc_mesh = plsc.VectorSubcoreMesh(core_axis_name="c", subcore_axis_name="s")
_sc_params = pltpu.CompilerParams(use_tc_tiling_on_sc=False)


@functools.partial(
    pl.kernel,
    out_type=jax.ShapeDtypeStruct((NC, ACC_ROWS, 16), jnp.float32),
    mesh=_sc_mesh,
    compiler_params=_sc_params,
    scratch_types=[
        pltpu.VMEM((NCH0, CHUNK), jnp.int32),
        pltpu.VMEM((CHUNK, 16), jnp.float32),
        pltpu.VMEM_SHARED((ACC_ROWS, 16), jnp.float32),
        pltpu.SemaphoreType.DMA,
    ],
)
def _sc_degree(dst_hbm, ones_hbm, zeros_hbm, out_hbm, didx_v, ones_v, acc,
               ssem):
    cid = lax.axis_index("c")
    sid = lax.axis_index("s")
    start = sid * STRIPE + cid * NCH0
    nch = jnp.where(cid == 0, NCH0, NCH1)

    row0 = sid * ROWS_PER_TILE
    pltpu.sync_copy(
        zeros_hbm.at[pl.ds(row0, ROWS_PER_TILE)],
        acc.at[pl.ds(row0, ROWS_PER_TILE)],
    )

    @pl.when(cid == 0)
    def _():
        pltpu.sync_copy(dst_hbm.at[pl.ds(start, NCH0)],
                        didx_v.at[pl.ds(0, NCH0)])

    @pl.when(cid == 1)
    def _():
        pltpu.sync_copy(dst_hbm.at[pl.ds(start, NCH1)],
                        didx_v.at[pl.ds(0, NCH1)])

    pltpu.sync_copy(ones_hbm, ones_v)
    plsc.subcore_barrier()

    def body(i, carry):
        pltpu.async_copy(ones_v, acc.at[didx_v.at[i]], ssem, add=True)
        return carry

    lax.fori_loop(0, nch, body, 0)

    def drain(i, carry):
        pltpu.make_async_copy(
            zeros_hbm.at[pl.ds(0, CHUNK)], ones_v, ssem
        ).wait()
        return carry

    lax.fori_loop(0, nch, drain, 0)
    plsc.subcore_barrier()
    pltpu.sync_copy(
        acc.at[pl.ds(row0, ROWS_PER_TILE)],
        out_hbm.at[cid, pl.ds(row0, ROWS_PER_TILE)],
    )


@functools.partial(
    pl.kernel,
    out_type=jax.ShapeDtypeStruct((NC, ACC_ROWS, D_HID), jnp.float32),
    mesh=_sc_mesh,
    compiler_params=_sc_params,
    scratch_types=[
        pltpu.VMEM((NCH0, CHUNK), jnp.int32),
        pltpu.VMEM((NCH0, CHUNK), jnp.int32),
        pltpu.VMEM((NBUF, CHUNK, D_HID), jnp.float32),
        pltpu.SemaphoreType.DMA,
        pltpu.SemaphoreType.DMA,
        pltpu.VMEM_SHARED((ACC_ROWS, D_HID), jnp.float32),
    ],
)
def _sc_agg(g_hbm, src_hbm, dst_hbm, zeros_hbm, out_hbm, sidx_v, didx_v,
            msg_v, gsem, ssem, acc):
    cid = lax.axis_index("c")
    sid = lax.axis_index("s")
    start = sid * STRIPE + cid * NCH0
    ng = jnp.where(cid == 0, NCH0 // K, NCH1 // K)

    row0 = sid * ROWS_PER_TILE
    pltpu.sync_copy(
        zeros_hbm.at[pl.ds(row0, ROWS_PER_TILE)],
        acc.at[pl.ds(row0, ROWS_PER_TILE)],
    )

    @pl.when(cid == 0)
    def _():
        pltpu.sync_copy(src_hbm.at[pl.ds(start, NCH0)],
                        sidx_v.at[pl.ds(0, NCH0)])
        pltpu.sync_copy(dst_hbm.at[pl.ds(start, NCH0)],
                        didx_v.at[pl.ds(0, NCH0)])

    @pl.when(cid == 1)
    def _():
        pltpu.sync_copy(src_hbm.at[pl.ds(start, NCH1)],
                        sidx_v.at[pl.ds(0, NCH1)])
        pltpu.sync_copy(dst_hbm.at[pl.ds(start, NCH1)],
                        didx_v.at[pl.ds(0, NCH1)])

    plsc.subcore_barrier()

    for j in range(2 * K):
        pltpu.async_copy(g_hbm.at[sidx_v.at[j]], msg_v.at[j], gsem)

    def body(g, carry):
        boff = (g % 2) * K
        for k in range(K):
            pltpu.make_async_copy(
                g_hbm.at[pl.ds(0, CHUNK)], msg_v.at[boff + k], gsem
            ).wait()
        sd = [
            pltpu.async_copy(msg_v.at[boff + k],
                             acc.at[didx_v.at[g * K + k]], ssem, add=True)
            for k in range(K)
        ]

        @pl.when(g + 2 < ng)
        def _():
            for k in range(K):
                sd[k].wait()
                pltpu.async_copy(g_hbm.at[sidx_v.at[(g + 2) * K + k]],
                                 msg_v.at[boff + k], gsem)
        return carry

    lax.fori_loop(0, ng, body, 0)
    for j in range(2 * K):
        pltpu.make_async_copy(
            g_hbm.at[pl.ds(0, CHUNK)], msg_v.at[j], ssem
        ).wait()
    plsc.subcore_barrier()
    pltpu.sync_copy(
        acc.at[pl.ds(row0, ROWS_PER_TILE)],
        out_hbm.at[cid, pl.ds(row0, ROWS_PER_TILE)],
    )


P_NODES = N_NODES * D_HID // 128
P_ACC = ACC_ROWS * D_HID // 128
P_BLK = P_NODES
GRID = 1
X_COLS = 8 * D_IN
W2_COLS = 8 * D_OUT


def _tc1a_body(x_ref, w1k_ref, h_ref):
    h_ref[...] = jnp.dot(x_ref[...], w1k_ref[...],
                         preferred_element_type=jnp.float32)


def _tc1a(xp, w1k):
    return pl.pallas_call(
        _tc1a_body,
        grid=(GRID,),
        in_specs=[
            pl.BlockSpec((P_NODES, X_COLS), lambda i: (0, 0)),
            pl.BlockSpec((X_COLS, 128), lambda i: (0, 0)),
        ],
        out_specs=pl.BlockSpec((P_NODES, 128), lambda i: (0, 0)),
        out_shape=jax.ShapeDtypeStruct((P_NODES, 128), jnp.float32),
    )(xp, w1k)


def _tc1b_body(h_ref, degp_ref, g1_ref, dinv_ref):
    deg = degp_ref[0, :P_NODES] + degp_ref[1, :P_NODES] + 1.0
    dinv = lax.rsqrt(deg)
    g1_ref[...] = h_ref[...] * dinv
    dinv_ref[...] = dinv


def _tc1b(h, degp):
    return pl.pallas_call(
        _tc1b_body,
        grid=(GRID,),
        in_specs=[
            pl.BlockSpec((P_NODES, 128), lambda i: (0, 0)),
            pl.BlockSpec((NC, P_ACC, 128), lambda i: (0, 0, 0)),
        ],
        out_specs=[
            pl.BlockSpec((P_NODES, 128), lambda i: (0, 0)),
            pl.BlockSpec((P_NODES, 128), lambda i: (0, 0)),
        ],
        out_shape=[
            jax.ShapeDtypeStruct((P_NODES, 128), jnp.float32),
            jax.ShapeDtypeStruct((P_NODES, 128), jnp.float32),
        ],
    )(h, degp)


def _tc2_body(aggp_ref, g1_ref, dinv_ref, b1_ref, g2_ref):
    s = aggp_ref[0, :P_NODES] + aggp_ref[1, :P_NODES] + g1_ref[...]
    out1 = jnp.maximum(dinv_ref[...] * s + b1_ref[...], 0.0)
    g2_ref[...] = out1 * dinv_ref[...]


def _tc2(aggp, g1, dinv, b1t):
    return pl.pallas_call(
        _tc2_body,
        grid=(GRID,),
        in_specs=[
            pl.BlockSpec((NC, P_ACC, 128), lambda i: (0, 0, 0)),
            pl.BlockSpec((P_NODES, 128), lambda i: (0, 0)),
            pl.BlockSpec((P_NODES, 128), lambda i: (0, 0)),
            pl.BlockSpec((1, 128), lambda i: (0, 0)),
        ],
        out_specs=pl.BlockSpec((P_NODES, 128), lambda i: (0, 0)),
        out_shape=jax.ShapeDtypeStruct((P_NODES, 128), jnp.float32),
    )(aggp, g1, dinv, b1t)


def _tc3_body(aggp_ref, g2_ref, dinv_ref, w2k_ref, b2_ref, out_ref):
    s = dinv_ref[...] * (
        aggp_ref[0, :P_NODES] + aggp_ref[1, :P_NODES] + g2_ref[...])
    out_ref[...] = (
        jnp.dot(s, w2k_ref[...], preferred_element_type=jnp.float32)
        + b2_ref[...]
    )


def _tc3(aggp, g2, dinv, w2k, b2t):
    return pl.pallas_call(
        _tc3_body,
        grid=(GRID,),
        in_specs=[
            pl.BlockSpec((NC, P_ACC, 128), lambda i: (0, 0, 0)),
            pl.BlockSpec((P_NODES, 128), lambda i: (0, 0)),
            pl.BlockSpec((P_NODES, 128), lambda i: (0, 0)),
            pl.BlockSpec((128, W2_COLS), lambda i: (0, 0)),
            pl.BlockSpec((1, W2_COLS), lambda i: (0, 0)),
        ],
        out_specs=pl.BlockSpec((P_NODES, W2_COLS), lambda i: (0, 0)),
        out_shape=jax.ShapeDtypeStruct((P_NODES, W2_COLS), jnp.float32),
    )(aggp, g2, dinv, w2k, b2t)


def kernel(x, edge_index, W1, b1, W2, b2):
    src = edge_index[0].astype(jnp.int32)
    dst = edge_index[1].astype(jnp.int32)
    npad = E_PAD - N_EDGES
    srcp = jnp.concatenate([src, jnp.zeros((npad,), jnp.int32)])
    srcp = srcp.reshape(N_CHUNKS, CHUNK)
    dstp = jnp.concatenate([dst, jnp.full((npad,), PAD_DST, jnp.int32)])
    dstp = dstp.reshape(N_CHUNKS, CHUNK)

    ones16 = jnp.ones((CHUNK, 16), jnp.float32)
    zeros16 = jnp.zeros((ACC_ROWS, 16), jnp.float32)

    eye8 = jnp.eye(8, dtype=jnp.float32)
    w1k = jnp.kron(eye8, W1)
    w2k = jnp.kron(eye8, W2)
    b1t = jnp.tile(b1, 8).reshape(1, 128)
    b2t = jnp.tile(b2, 8).reshape(1, W2_COLS)

    degp = _sc_degree(dstp, ones16, zeros16)
    degp_p = degp.reshape(NC, P_ACC, 128)

    xp = x.reshape(P_NODES, X_COLS)
    h1 = _tc1a(xp, w1k)
    g1, dinv = _tc1b(h1, degp_p)

    g1u = g1.reshape(N_NODES, D_HID)
    agg1 = _sc_agg(g1u, srcp, dstp, zeros16)
    g2 = _tc2(agg1.reshape(NC, P_ACC, 128), g1, dinv, b1t)

    g2u = g2.reshape(N_NODES, D_HID)
    agg2 = _sc_agg(g2u, srcp, dstp, zeros16)
    outp = _tc3(agg2.reshape(NC, P_ACC, 128), g2, dinv, w2k, b2t)

    return outp.reshape(N_NODES, D_OUT)


# speedup vs baseline: 86.6753x; 1.6163x over previous
import functools

import jax
import jax.numpy as jnp
from jax import lax
from jax.experimental import pallas as pl
from jax.experimental.pallas import tpu as pltpu
from jax.experimental.pallas import tpu_sc as plsc

N_NODES = 10000
N_EDGES = 320000
D_IN = 128
D_HID = 16
D_OUT = 40

NC = 2
NS = 16

CHUNK = 128
N_CHUNKS = N_EDGES // CHUNK
NSLAB = 120
NBUF = 16
D_LEAD = 8
S_LAG = 8
R_NUM, R_DEN = 13, 20
ACC_ROWS = 10112
ROWS_PER_TILE = ACC_ROWS // NS

_sc_mesh = plsc.VectorSubcoreMesh(core_axis_name="c", subcore_axis_name="s")
_sc_params = pltpu.CompilerParams(use_tc_tiling_on_sc=False)


def _worker_range(cid, sid):
    ps = sid * N_CHUNKS // NS
    pe = (sid + 1) * N_CHUNKS // NS
    n_pair = pe - ps
    n0 = n_pair * R_NUM // R_DEN
    start = ps + cid * n0
    n = jnp.where(cid == 0, n0, n_pair - n0)
    sstart = jnp.minimum(start, N_CHUNKS - NSLAB)
    return sstart, start - sstart, n


@functools.partial(
    pl.kernel,
    out_type=jax.ShapeDtypeStruct((NC, ACC_ROWS, 16), jnp.float32),
    mesh=_sc_mesh,
    compiler_params=_sc_params,
    scratch_types=[
        pltpu.VMEM((NSLAB, CHUNK), jnp.int32),
        pltpu.VMEM((CHUNK, 16), jnp.float32),
        pltpu.VMEM_SHARED((ACC_ROWS, 16), jnp.float32),
        pltpu.SemaphoreType.DMA,
    ],
)
def _sc_degree(ei_hbm, ones_hbm, zeros_hbm, out_hbm, didx_v, ones_v, acc,
               ssem):
    cid = lax.axis_index("c")
    sid = lax.axis_index("s")
    sstart, off, nch = _worker_range(cid, sid)

    row0 = sid * ROWS_PER_TILE
    pltpu.sync_copy(
        zeros_hbm.at[pl.ds(row0, ROWS_PER_TILE)],
        acc.at[pl.ds(row0, ROWS_PER_TILE)],
    )
    pltpu.sync_copy(ei_hbm.at[1, pl.ds(sstart, NSLAB)], didx_v)
    pltpu.sync_copy(ones_hbm, ones_v)
    plsc.subcore_barrier()

    def body(i, carry):
        pltpu.async_copy(ones_v, acc.at[didx_v.at[off + i]], ssem, add=True)
        return carry

    lax.fori_loop(0, nch, body, 0)

    def drain(i, carry):
        pltpu.make_async_copy(
            zeros_hbm.at[pl.ds(0, CHUNK)], ones_v, ssem
        ).wait()
        return carry

    lax.fori_loop(0, nch, drain, 0)
    plsc.subcore_barrier()
    pltpu.sync_copy(
        acc.at[pl.ds(row0, ROWS_PER_TILE)],
        out_hbm.at[cid, pl.ds(row0, ROWS_PER_TILE)],
    )


@functools.partial(
    pl.kernel,
    out_type=jax.ShapeDtypeStruct((NC, ACC_ROWS, D_HID), jnp.float32),
    mesh=_sc_mesh,
    compiler_params=_sc_params,
    scratch_types=[
        pltpu.VMEM((NSLAB, CHUNK), jnp.int32),
        pltpu.VMEM((NSLAB, CHUNK), jnp.int32),
        pltpu.VMEM((NBUF, CHUNK, D_HID), jnp.float32),
        pltpu.SemaphoreType.DMA,
        pltpu.SemaphoreType.DMA,
        pltpu.VMEM_SHARED((ACC_ROWS, D_HID), jnp.float32),
    ],
)
def _sc_agg(g_hbm, ei_hbm, zeros_hbm, out_hbm, sidx_v, didx_v, msg_v, gsem,
            ssem, acc):
    cid = lax.axis_index("c")
    sid = lax.axis_index("s")
    sstart, off, nch = _worker_range(cid, sid)

    row0 = sid * ROWS_PER_TILE
    pltpu.sync_copy(
        zeros_hbm.at[pl.ds(row0, ROWS_PER_TILE)],
        acc.at[pl.ds(row0, ROWS_PER_TILE)],
    )
    pltpu.sync_copy(ei_hbm.at[0, pl.ds(sstart, NSLAB)], sidx_v)
    pltpu.sync_copy(ei_hbm.at[1, pl.ds(sstart, NSLAB)], didx_v)
    plsc.subcore_barrier()

    def prologue(j, carry):
        pltpu.async_copy(g_hbm.at[sidx_v.at[off + j]], msg_v.at[j], gsem)
        return carry

    lax.fori_loop(0, jnp.minimum(D_LEAD, nch), prologue, 0)

    def body(i, carry):
        b = lax.rem(i, NBUF)
        pltpu.make_async_copy(
            g_hbm.at[pl.ds(0, CHUNK)], msg_v.at[b], gsem
        ).wait()
        pltpu.async_copy(msg_v.at[b], acc.at[didx_v.at[off + i]], ssem,
                         add=True)

        @pl.when(i >= S_LAG)
        def _():
            pltpu.make_async_copy(
                g_hbm.at[pl.ds(0, CHUNK)], msg_v.at[b], ssem
            ).wait()

        @pl.when(i + D_LEAD < nch)
        def _():
            pltpu.async_copy(g_hbm.at[sidx_v.at[off + i + D_LEAD]],
                             msg_v.at[lax.rem(i + D_LEAD, NBUF)], gsem)
        return carry

    lax.fori_loop(0, nch, body, 0)

    def drain(j, carry):
        pltpu.make_async_copy(
            g_hbm.at[pl.ds(0, CHUNK)], msg_v.at[0], ssem
        ).wait()
        return carry

    lax.fori_loop(0, jnp.minimum(S_LAG, nch), drain, 0)
    plsc.subcore_barrier()
    pltpu.sync_copy(
        acc.at[pl.ds(row0, ROWS_PER_TILE)],
        out_hbm.at[cid, pl.ds(row0, ROWS_PER_TILE)],
    )


P_NODES = N_NODES * D_HID // 128
P_ACC = ACC_ROWS * D_HID // 128
X_COLS = 8 * D_IN
W2_COLS = 8 * D_OUT


def _tc1a_body(x_ref, w1k_ref, h_ref):
    h_ref[...] = jnp.dot(x_ref[...], w1k_ref[...],
                         preferred_element_type=jnp.float32)


def _tc1a(xp, w1k):
    return pl.pallas_call(
        _tc1a_body,
        in_specs=[
            pl.BlockSpec((P_NODES, X_COLS), lambda: (0, 0)),
            pl.BlockSpec((X_COLS, 128), lambda: (0, 0)),
        ],
        out_specs=pl.BlockSpec((P_NODES, 128), lambda: (0, 0)),
        out_shape=jax.ShapeDtypeStruct((P_NODES, 128), jnp.float32),
    )(xp, w1k)


def _tc1b_body(h_ref, degp_ref, g1_ref, dinv_ref):
    deg = degp_ref[0, :P_NODES] + degp_ref[1, :P_NODES] + 1.0
    dinv = lax.rsqrt(deg)
    g1_ref[...] = h_ref[...] * dinv
    dinv_ref[...] = dinv


def _tc1b(h, degp):
    return pl.pallas_call(
        _tc1b_body,
        in_specs=[
            pl.BlockSpec((P_NODES, 128), lambda: (0, 0)),
            pl.BlockSpec((NC, P_ACC, 128), lambda: (0, 0, 0)),
        ],
        out_specs=[
            pl.BlockSpec((P_NODES, 128), lambda: (0, 0)),
            pl.BlockSpec((P_NODES, 128), lambda: (0, 0)),
        ],
        out_shape=[
            jax.ShapeDtypeStruct((P_NODES, 128), jnp.float32),
            jax.ShapeDtypeStruct((P_NODES, 128), jnp.float32),
        ],
    )(h, degp)


def _tc2_body(aggp_ref, g1_ref, dinv_ref, b1_ref, g2_ref):
    s = aggp_ref[0, :P_NODES] + aggp_ref[1, :P_NODES] + g1_ref[...]
    out1 = jnp.maximum(dinv_ref[...] * s + b1_ref[...], 0.0)
    g2_ref[...] = out1 * dinv_ref[...]


def _tc2(aggp, g1, dinv, b1t):
    return pl.pallas_call(
        _tc2_body,
        in_specs=[
            pl.BlockSpec((NC, P_ACC, 128), lambda: (0, 0, 0)),
            pl.BlockSpec((P_NODES, 128), lambda: (0, 0)),
            pl.BlockSpec((P_NODES, 128), lambda: (0, 0)),
            pl.BlockSpec((1, 128), lambda: (0, 0)),
        ],
        out_specs=pl.BlockSpec((P_NODES, 128), lambda: (0, 0)),
        out_shape=jax.ShapeDtypeStruct((P_NODES, 128), jnp.float32),
    )(aggp, g1, dinv, b1t)


def _tc3_body(aggp_ref, g2_ref, dinv_ref, w2k_ref, b2_ref, out_ref):
    s = dinv_ref[...] * (
        aggp_ref[0, :P_NODES] + aggp_ref[1, :P_NODES] + g2_ref[...])
    out_ref[...] = (
        jnp.dot(s, w2k_ref[...], preferred_element_type=jnp.float32)
        + b2_ref[...]
    )


def _tc3(aggp, g2, dinv, w2k, b2t):
    return pl.pallas_call(
        _tc3_body,
        in_specs=[
            pl.BlockSpec((NC, P_ACC, 128), lambda: (0, 0, 0)),
            pl.BlockSpec((P_NODES, 128), lambda: (0, 0)),
            pl.BlockSpec((P_NODES, 128), lambda: (0, 0)),
            pl.BlockSpec((128, W2_COLS), lambda: (0, 0)),
            pl.BlockSpec((1, W2_COLS), lambda: (0, 0)),
        ],
        out_specs=pl.BlockSpec((P_NODES, W2_COLS), lambda: (0, 0)),
        out_shape=jax.ShapeDtypeStruct((P_NODES, W2_COLS), jnp.float32),
    )(aggp, g2, dinv, w2k, b2t)


def kernel(x, edge_index, W1, b1, W2, b2):
    ei3 = edge_index.astype(jnp.int32).reshape(2, N_CHUNKS, CHUNK)

    ones16 = jnp.ones((CHUNK, 16), jnp.float32)
    zeros16 = jnp.zeros((ACC_ROWS, 16), jnp.float32)

    eye8 = jnp.eye(8, dtype=jnp.float32)
    w1k = jnp.kron(eye8, W1)
    w2k = jnp.kron(eye8, W2)
    b1t = jnp.tile(b1, 8).reshape(1, 128)
    b2t = jnp.tile(b2, 8).reshape(1, W2_COLS)

    degp = _sc_degree(ei3, ones16, zeros16)
    degp_p = degp.reshape(NC, P_ACC, 128)

    xp = x.reshape(P_NODES, X_COLS)
    h1 = _tc1a(xp, w1k)
    g1, dinv = _tc1b(h1, degp_p)

    g1u = g1.reshape(N_NODES, D_HID)
    agg1 = _sc_agg(g1u, ei3, zeros16)
    g2 = _tc2(agg1.reshape(NC, P_ACC, 128), g1, dinv, b1t)

    g2u = g2.reshape(N_NODES, D_HID)
    agg2 = _sc_agg(g2u, ei3, zeros16)
    outp = _tc3(agg2.reshape(NC, P_ACC, 128), g2, dinv, w2k, b2t)

    return outp.reshape(N_NODES, D_OUT)
